# baseline (speedup 1.0000x reference)
import functools
import jax
import jax.numpy as jnp
from jax import lax
from jax.experimental import pallas as pl
from jax.experimental.pallas import tpu as pltpu
from jax.experimental.pallas import tpu_sc as plsc

_N = 16384
_K = 8192
_PER_TILE = 1024
_NV = _PER_TILE // 16
_INT_MIN = -(2**31)
_UNROLL = 4

_mesh = plsc.VectorSubcoreMesh(core_axis_name="c", subcore_axis_name="s")


@functools.partial(
    pl.kernel,
    mesh=_mesh,
    out_type=jax.ShapeDtypeStruct((16,), jnp.float32),
    scratch_types=[
        pltpu.VMEM((_PER_TILE,), jnp.float32),
        pltpu.VMEM((_PER_TILE,), jnp.int32),
        pltpu.VMEM((16,), jnp.int32),
        pltpu.VMEM((16,), jnp.int32),
        pltpu.VMEM((48,), jnp.int32),
        pltpu.VMEM((16,), jnp.int32),
        pltpu.VMEM((16,), jnp.float32),
        pltpu.VMEM((256,), jnp.float32),
        pltpu.VMEM((16,), jnp.float32),
        pltpu.VMEM_SHARED((640,), jnp.int32),
        pltpu.VMEM_SHARED((256,), jnp.float32),
    ],
)
def _sc_select_mean(
    batch_hbm, out_hbm, xv, skv, accv, idxv, zv, rdv, psv, frd, obuf, hist, fsum
):
    c = lax.axis_index("c")
    s = lax.axis_index("s")
    iota = lax.iota(jnp.int32, 16)
    zero16i = jnp.zeros((16,), jnp.int32)
    zero16f = jnp.zeros((16,), jnp.float32)

    pltpu.sync_copy(batch_hbm.at[pl.ds(s * _PER_TILE, _PER_TILE)], xv)
    for j in range(3):
        zv[pl.ds(j * 16, 16)] = zero16i

    def _mkkeys(i, carry):
        x = xv[pl.ds(i * 16, 16)]
        b = lax.bitcast_convert_type(x, jnp.int32)
        sk = jnp.where(b >= 0, b, b ^ jnp.int32(0x7FFFFFFF))
        skv[pl.ds(i * 16, 16)] = sk
        return carry

    lax.fori_loop(0, _NV, _mkkeys, jnp.int32(0))

    pltpu.sync_copy(zv.at[pl.ds(0, 40)], hist.at[pl.ds(s * 40, 40)])
    plsc.subcore_barrier()

    cand = jnp.int32(0)
    for p in range(32):
        bit = jnp.int32((1 << (31 - p)) - (1 << 32 if p == 0 else 0))
        cand_try = cand | bit
        thr = cand_try ^ jnp.int32(_INT_MIN)
        thv = jnp.broadcast_to(thr, (16,))

        def _count(i, acc, _thv=thv):
            a = acc
            for u in range(_UNROLL):
                sk = skv[pl.ds((i * _UNROLL + u) * 16, 16)]
                a = a + jnp.where(sk >= _thv, jnp.int32(1), jnp.int32(0))
            return a

        acc = lax.fori_loop(0, _NV // _UNROLL, _count, zero16i)
        accv[...] = acc
        idxv[...] = iota + jnp.int32(p * 16)
        pltpu.sync_copy(accv, hist.at[idxv], add=True)
        plsc.subcore_barrier()
        pltpu.sync_copy(hist.at[pl.ds(p * 16, 16)], rdv)
        v = rdv[...]
        tot = v[0]
        for l in range(1, 16):
            tot = tot + v[l]
        cand = jnp.where(tot >= jnp.int32(_K), cand_try, cand)

    t_signed = cand ^ jnp.int32(_INT_MIN)
    tv = jnp.broadcast_to(t_signed, (16,))

    def _fin(i, carry):
        ps, ac = carry
        for u in range(_UNROLL):
            sl = pl.ds((i * _UNROLL + u) * 16, 16)
            sk = skv[sl]
            x = xv[sl]
            gt = sk > tv
            ps = ps + jnp.where(gt, x, jnp.float32(0.0))
            ac = ac + jnp.where(gt, jnp.int32(1), jnp.int32(0))
        return ps, ac

    psum, acc = lax.fori_loop(0, _NV // _UNROLL, _fin, (zero16f, zero16i))
    accv[...] = acc
    idxv[...] = iota + jnp.int32(512)
    pltpu.sync_copy(accv, hist.at[idxv], add=True)
    psv[...] = psum
    pltpu.sync_copy(psv, fsum.at[pl.ds(s * 16, 16)])
    plsc.subcore_barrier()

    @pl.when(jnp.logical_and(c == 0, s == 0))
    def _():
        pltpu.sync_copy(hist.at[pl.ds(512, 16)], rdv)
        v = rdv[...]
        cnt_gt = v[0]
        for l in range(1, 16):
            cnt_gt = cnt_gt + v[l]
        pltpu.sync_copy(fsum, frd)
        pv = frd[pl.ds(0, 16)]
        for t in range(1, 16):
            pv = pv + frd[pl.ds(t * 16, 16)]
        s_gt = pv[0]
        for l in range(1, 16):
            s_gt = s_gt + pv[l]
        needv = jnp.broadcast_to(jnp.int32(_K) - cnt_gt, (16,)).astype(jnp.float32)
        bv = jnp.where(tv >= 0, tv, tv ^ jnp.int32(0x7FFFFFFF))
        valv = lax.bitcast_convert_type(bv, jnp.float32)
        outv = (jnp.broadcast_to(s_gt, (16,)) + needv * valv) * jnp.float32(1.0 / _K)
        obuf[...] = outv
        pltpu.sync_copy(obuf, out_hbm)


def sc_kernel(batch):
    out = _sc_select_mean(batch)
    return out[0]


def kernel(batch):
    return sc_kernel(batch)


# speedup vs baseline: 11.1839x; 11.1839x over previous
import jax
import jax.numpy as jnp
from jax import lax
from jax.experimental import pallas as pl

_N = 16384
_K = 8192
_INT_MIN = -(2**31)


def _onehot_words(nib):
    one = jnp.int32(1)
    amt = lax.shift_left(nib & jnp.int32(7), jnp.int32(2))
    w = lax.shift_left(one, amt)
    zero = jnp.int32(0)
    wa = jnp.where(nib < 8, w, zero)
    wb = jnp.where(nib >= 8, w, zero)
    return wa, wb


def _hist16(wa, wb, act):
    zero = jnp.int32(0)
    words = []
    for wfull in (wa, wb):
        wv = wfull if act is None else jnp.where(act, wfull, zero)
        h1 = wv[0:8] + wv[8:16]
        h2 = wv[16:24] + wv[24:32]
        h3 = wv[32:40] + wv[40:48]
        h4 = wv[48:56] + wv[56:64]
        q1 = h1 + h2
        q2 = h3 + h4
        a1 = q1 + q2
        h5 = wv[64:72] + wv[72:80]
        h6 = wv[80:88] + wv[88:96]
        h7 = wv[96:104] + wv[104:112]
        h8 = wv[112:120] + wv[120:128]
        q3 = h5 + h6
        q4 = h7 + h8
        a2 = q3 + q4
        mask4 = jnp.int32(0x0F0F0F0F)
        ev = (a1 & mask4) + (a2 & mask4)
        od = (lax.shift_right_logical(a1, jnp.int32(4)) & mask4) + (
            lax.shift_right_logical(a2, jnp.int32(4)) & mask4
        )
        ev = jnp.sum(ev, axis=0, keepdims=True)
        od = jnp.sum(od, axis=0, keepdims=True)
        mask8 = jnp.int32(0x00FF00FF)
        rows = [
            ev & mask8,
            lax.shift_right_logical(ev, jnp.int32(8)) & mask8,
            od & mask8,
            lax.shift_right_logical(od, jnp.int32(8)) & mask8,
        ]
        words.append([jnp.sum(r) for r in rows])

    mask16 = jnp.int32(0xFFFF)
    h = [None] * 16
    for g, packs in enumerate(words):
        for r, s in enumerate(packs):
            lo_bin = (0, 2, 1, 3)[r]
            h[g * 8 + lo_bin] = s & mask16
            h[g * 8 + lo_bin + 4] = lax.shift_right_logical(s, jnp.int32(16))
    return h


def _select_mean_body(x_ref, o_ref):
    x = x_ref[...]
    b = lax.bitcast_convert_type(x, jnp.int32)
    sk = jnp.where(b >= 0, b, b ^ jnp.int32(0x7FFFFFFF))
    uk = sk ^ jnp.int32(_INT_MIN)

    prefix = jnp.int32(0)
    need = jnp.int32(_K)
    nib0 = lax.shift_right_logical(uk, jnp.int32(28))
    nxt = _onehot_words(nib0)
    for p in range(8):
        wa, wb = nxt
        if p == 0:
            act = None
        else:
            shift = 28 - 4 * p
            hi = lax.shift_right_logical(uk, jnp.int32(shift + 4))
            act = hi == prefix
        if p < 7:
            nshift = 28 - 4 * (p + 1)
            nnib = lax.shift_right_logical(uk, jnp.int32(nshift)) & jnp.int32(15)
            nxt = _onehot_words(nnib)
        h = _hist16(wa, wb, act)
        sfx = [None] * 16
        run = h[15]
        sfx[15] = run
        for v in range(14, 0, -1):
            run = run + h[v]
            sfx[v] = run
        zero = jnp.int32(0)
        bstar = zero
        cnt_above = zero
        for v in range(1, 16):
            bstar = bstar + jnp.where(sfx[v] >= need, jnp.int32(1), zero)
            cnt_above = jnp.maximum(
                cnt_above, jnp.where(sfx[v] < need, sfx[v], zero)
            )
        need = need - cnt_above
        prefix = (prefix << jnp.int32(4)) | bstar

    t_sk = prefix ^ jnp.int32(_INT_MIN)
    gt = sk > t_sk
    sum_gt = jnp.sum(jnp.where(gt, x, jnp.float32(0.0)))
    tb_ = jnp.where(t_sk >= 0, t_sk, t_sk ^ jnp.int32(0x7FFFFFFF))
    val_t = lax.bitcast_convert_type(tb_, jnp.float32)
    out = (sum_gt + need.astype(jnp.float32) * val_t) / jnp.float32(_K)
    o_ref[...] = jnp.full((1, 1), out, jnp.float32)


def kernel(batch):
    x2d = batch.reshape(128, 128)
    out = pl.pallas_call(
        _select_mean_body,
        out_shape=jax.ShapeDtypeStruct((1, 1), jnp.float32),
    )(x2d)
    return out.reshape(())


# speedup vs baseline: 11.8766x; 1.0619x over previous
import jax
import jax.numpy as jnp
from jax import lax
from jax.experimental import pallas as pl

_N = 16384
_K = 8192
_INT_MIN = -(2**31)


def _onehot_words(nib):
    one = jnp.int32(1)
    amt = lax.shift_left(nib & jnp.int32(7), jnp.int32(2))
    w = lax.shift_left(one, amt)
    zero = jnp.int32(0)
    wa = jnp.where(nib < 8, w, zero)
    wb = jnp.where(nib >= 8, w, zero)
    return wa, wb


def _hist16(wa, wb, act):
    zero = jnp.int32(0)
    words = []
    for wfull in (wa, wb):
        wv = wfull if act is None else jnp.where(act, wfull, zero)
        h1 = wv[0:8] + wv[8:16]
        h2 = wv[16:24] + wv[24:32]
        h3 = wv[32:40] + wv[40:48]
        h4 = wv[48:56] + wv[56:64]
        q1 = h1 + h2
        q2 = h3 + h4
        a1 = q1 + q2
        h5 = wv[64:72] + wv[72:80]
        h6 = wv[80:88] + wv[88:96]
        h7 = wv[96:104] + wv[104:112]
        h8 = wv[112:120] + wv[120:128]
        q3 = h5 + h6
        q4 = h7 + h8
        a2 = q3 + q4
        mask4 = jnp.int32(0x0F0F0F0F)
        ev = (a1 & mask4) + (a2 & mask4)
        od = (lax.shift_right_logical(a1, jnp.int32(4)) & mask4) + (
            lax.shift_right_logical(a2, jnp.int32(4)) & mask4
        )
        ev = jnp.sum(ev, axis=0, keepdims=True)
        od = jnp.sum(od, axis=0, keepdims=True)
        mask8 = jnp.int32(0x00FF00FF)
        rows = [
            ev & mask8,
            lax.shift_right_logical(ev, jnp.int32(8)) & mask8,
            od & mask8,
            lax.shift_right_logical(od, jnp.int32(8)) & mask8,
        ]
        words.append([jnp.sum(r) for r in rows])

    mask16 = jnp.int32(0xFFFF)
    h = [None] * 16
    for g, packs in enumerate(words):
        for r, s in enumerate(packs):
            lo_bin = (0, 2, 1, 3)[r]
            h[g * 8 + lo_bin] = s & mask16
            h[g * 8 + lo_bin + 4] = lax.shift_right_logical(s, jnp.int32(16))
    return h


def _select_mean_body(x_ref, o_ref):
    x = x_ref[...]
    b = lax.bitcast_convert_type(x, jnp.int32)
    sk = jnp.where(b >= 0, b, b ^ jnp.int32(0x7FFFFFFF))
    uk = sk ^ jnp.int32(_INT_MIN)

    prefix = jnp.int32(0)
    need = jnp.int32(_K)
    nib0 = lax.shift_right_logical(uk, jnp.int32(28))
    nxt = _onehot_words(nib0)
    for p in range(8):
        wa, wb = nxt
        if p == 0:
            act = None
        else:
            shift = 28 - 4 * p
            hi = lax.shift_right_logical(uk, jnp.int32(shift + 4))
            act = hi == prefix
        if p < 7:
            nshift = 28 - 4 * (p + 1)
            nnib = lax.shift_right_logical(uk, jnp.int32(nshift)) & jnp.int32(15)
            nxt = _onehot_words(nnib)
        h = _hist16(wa, wb, act)
        pr = [h[2 * i] + h[2 * i + 1] for i in range(8)]
        q = [pr[2 * i] + pr[2 * i + 1] for i in range(4)]
        zero = jnp.int32(0)
        a1 = q[2] + q[3]
        t1 = a1 >= need
        above = jnp.where(t1, zero, a1)
        c2 = jnp.where(t1, q[3], q[1])
        t2 = above + c2 >= need
        above = jnp.where(t2, above, above + c2)
        c3 = jnp.where(
            t1, jnp.where(t2, pr[7], pr[5]), jnp.where(t2, pr[3], pr[1])
        )
        t3 = above + c3 >= need
        above = jnp.where(t3, above, above + c3)
        c4 = jnp.where(
            t1,
            jnp.where(t2, jnp.where(t3, h[15], h[13]), jnp.where(t3, h[11], h[9])),
            jnp.where(t2, jnp.where(t3, h[7], h[5]), jnp.where(t3, h[3], h[1])),
        )
        t4 = above + c4 >= need
        above = jnp.where(t4, above, above + c4)
        bstar = (
            jnp.where(t1, jnp.int32(8), zero)
            + jnp.where(t2, jnp.int32(4), zero)
            + jnp.where(t3, jnp.int32(2), zero)
            + jnp.where(t4, jnp.int32(1), zero)
        )
        need = need - above
        prefix = (prefix << jnp.int32(4)) | bstar

    t_sk = prefix ^ jnp.int32(_INT_MIN)
    gt = sk > t_sk
    sum_gt = jnp.sum(jnp.where(gt, x, jnp.float32(0.0)))
    tb_ = jnp.where(t_sk >= 0, t_sk, t_sk ^ jnp.int32(0x7FFFFFFF))
    val_t = lax.bitcast_convert_type(tb_, jnp.float32)
    out = (sum_gt + need.astype(jnp.float32) * val_t) / jnp.float32(_K)
    o_ref[...] = jnp.full((1, 1), out, jnp.float32)


def kernel(batch):
    x2d = batch.reshape(128, 128)
    out = pl.pallas_call(
        _select_mean_body,
        out_shape=jax.ShapeDtypeStruct((1, 1), jnp.float32),
    )(x2d)
    return out.reshape(())
